# baseline (speedup 1.0000x reference)
import jax
import jax.numpy as jnp
from jax import lax
from jax.experimental import pallas as pl
from jax.experimental.pallas import tpu as pltpu
from jax.experimental.pallas import tpu_sc as plsc

_N = 10000
_H = 128
_NC = 2
_NS = 16
_NW = _NC * _NS
_CHUNK = 128
_NACC = 10112
_ZR = _NACC // _NS


def _sc_segment_sum(k0, k1, nch, dst_off, h_hbm, edges_hbm, out0_hbm, out1_hbm,
                    srcb0, dstb0, srcb1, dstb1, rows0, rows1,
                    acc, semi0, semi1, semg0, semg1, sems0):
    c = lax.axis_index("c")
    s = lax.axis_index("s")
    is0 = c == 0
    my_k = jnp.where(is0, k0, k1)
    base = jnp.where(is0, s * k0, _NS * k0 + s * k1)
    my_k = jnp.minimum(my_k, jnp.maximum(nch - base, 0))

    def zbody(r, carry):
        for q in range(8):
            rows0[r, pl.ds(q * 16, 16)] = jnp.zeros((16,), jnp.float32)
        return carry

    lax.fori_loop(0, _CHUNK, zbody, 0)
    for t in range(4):
        pltpu.sync_copy(rows0, acc.at[pl.ds(s * _ZR + t * _CHUNK, _CHUNK)])
    pltpu.sync_copy(rows0.at[pl.ds(0, _ZR - 4 * _CHUNK)],
                    acc.at[pl.ds(s * _ZR + 4 * _CHUNK, _ZR - 4 * _CHUNK)])
    plsc.subcore_barrier()

    def sslice(g):
        return pl.ds(g * _CHUNK, _CHUNK)

    def dslice(g):
        return pl.ds(dst_off + g * _CHUNK, _CHUNK)

    @pl.when(my_k > 0)
    def _():
        pltpu.async_copy(edges_hbm.at[sslice(base)], srcb0, semi0)
        pltpu.async_copy(edges_hbm.at[dslice(base)], dstb0, semi0)
        pltpu.async_copy(edges_hbm.at[sslice(base + 1)], srcb1, semi1)
        pltpu.async_copy(edges_hbm.at[dslice(base + 1)], dstb1, semi1)

    def body(i, carry):
        a = 2 * i
        pltpu.make_async_copy(edges_hbm.at[sslice(base + a)], srcb0, semi0).wait()
        pltpu.make_async_copy(edges_hbm.at[dslice(base + a)], dstb0, semi0).wait()
        g0 = pltpu.async_copy(h_hbm.at[srcb0], rows0, semg0)
        pltpu.make_async_copy(edges_hbm.at[sslice(base + a + 1)], srcb1, semi1).wait()
        pltpu.make_async_copy(edges_hbm.at[dslice(base + a + 1)], dstb1, semi1).wait()
        g1 = pltpu.async_copy(h_hbm.at[srcb1], rows1, semg1)
        g0.wait()
        s0 = pltpu.async_copy(rows0, acc.at[dstb0], sems0, add=True)
        g1.wait()
        pltpu.sync_copy(rows1, acc.at[dstb1], add=True)
        s0.wait()

        @pl.when(a + 2 < my_k)
        def _():
            pltpu.async_copy(edges_hbm.at[sslice(base + a + 2)], srcb0, semi0)
            pltpu.async_copy(edges_hbm.at[dslice(base + a + 2)], dstb0, semi0)

        @pl.when(a + 3 < my_k)
        def _():
            pltpu.async_copy(edges_hbm.at[sslice(base + a + 3)], srcb1, semi1)
            pltpu.async_copy(edges_hbm.at[dslice(base + a + 3)], dstb1, semi1)

        return carry

    lax.fori_loop(0, my_k // 2, body, 0)
    plsc.subcore_barrier()

    rpw = (_N // _NS) & ~7
    tail = _N - _NS * rpw

    @pl.when(is0)
    def _():
        pltpu.sync_copy(acc.at[pl.ds(s * rpw, rpw)],
                        out0_hbm.at[pl.ds(s * rpw, rpw)])

        @pl.when(s == 0)
        def _():
            pltpu.sync_copy(acc.at[pl.ds(_NS * rpw, tail)],
                            out0_hbm.at[pl.ds(_NS * rpw, tail)])

    @pl.when(jnp.logical_not(is0))
    def _():
        pltpu.sync_copy(acc.at[pl.ds(s * rpw, rpw)],
                        out1_hbm.at[pl.ds(s * rpw, rpw)])

        @pl.when(s == 0)
        def _():
            pltpu.sync_copy(acc.at[pl.ds(_NS * rpw, tail)],
                            out1_hbm.at[pl.ds(_NS * rpw, tail)])


def _make_seg(k0, k1, nch, dst_off):
    import functools
    mesh = plsc.VectorSubcoreMesh(core_axis_name="c", subcore_axis_name="s")
    return pl.kernel(
        functools.partial(_sc_segment_sum, k0, k1, nch, dst_off),
        mesh=mesh,
        out_type=[jax.ShapeDtypeStruct((_N, _H), jnp.float32),
                  jax.ShapeDtypeStruct((_N, _H), jnp.float32)],
        scratch_types=[
            pltpu.VMEM((_CHUNK,), jnp.int32),
            pltpu.VMEM((_CHUNK,), jnp.int32),
            pltpu.VMEM((_CHUNK,), jnp.int32),
            pltpu.VMEM((_CHUNK,), jnp.int32),
            pltpu.VMEM((_CHUNK, _H), jnp.float32),
            pltpu.VMEM((_CHUNK, _H), jnp.float32),
            pltpu.VMEM_SHARED((_NACC, _H), jnp.float32),
            pltpu.SemaphoreType.DMA,
            pltpu.SemaphoreType.DMA,
            pltpu.SemaphoreType.DMA,
            pltpu.SemaphoreType.DMA,
            pltpu.SemaphoreType.DMA,
        ],
    )


def _dense(p0, p1, h, W, b, Wr, br, u_out, stats):
    i = pl.program_id(0)
    agg = p0[...] + p1[...]
    u = jnp.maximum(jnp.dot(agg, W[...], preferred_element_type=jnp.float32)
                    + b[...], 0.0)
    r = jnp.maximum(jnp.dot(h[...], Wr[...], preferred_element_type=jnp.float32)
                    + br[...], 0.0)
    u = u + r
    u_out[...] = u

    @pl.when(i == 0)
    def _():
        stats[...] = jnp.zeros_like(stats)

    stats[0:1, :] += jnp.sum(u, axis=0, keepdims=True)
    stats[1:2, :] += jnp.sum(u * u, axis=0, keepdims=True)


def _bn(u, stats, g, be, h_out):
    mu = stats[0:1, :] * (1.0 / _N)
    var = stats[1:2, :] * (1.0 / _N) - mu * mu
    sc = g[...] * lax.rsqrt(var + 1e-5)
    h_out[...] = (u[...] - mu) * sc + be[...]


def _bn_readout(u, stats, g, be, watt, batt, sum_out, max_out):
    i = pl.program_id(0)
    mu = stats[0:1, :] * (1.0 / _N)
    var = stats[1:2, :] * (1.0 / _N) - mu * mu
    sc = g[...] * lax.rsqrt(var + 1e-5)
    hh = (u[...] - mu) * sc + be[...]
    logits = jnp.dot(hh, watt[...], preferred_element_type=jnp.float32) + batt[...]
    w = jax.nn.sigmoid(logits[:, 0:1])
    ps = jnp.sum(w * hh, axis=0, keepdims=True)
    pm = jnp.max(hh, axis=0, keepdims=True)

    @pl.when(i == 0)
    def _():
        sum_out[...] = jnp.zeros_like(sum_out)
        max_out[...] = jnp.full_like(max_out, -jnp.inf)

    sum_out[0:1, :] += ps
    max_out[0:1, :] = jnp.maximum(max_out[0:1, :], pm)


def kernel(x, edge_index, W1, b1, Wr1, br1, g1, be1,
           W2, b2, Wr2, br2, g2, be2, w_att, b_att):
    E = edge_index.shape[1]
    nch = E // _CHUNK
    if nch * _CHUNK == E and nch % 2 == 0:
        edges = edge_index.reshape(2 * E)
        dst_off = E
    else:
        nch = -(-E // _CHUNK)
        nch += nch & 1
        padn = nch * _CHUNK - E
        pad_dst = _N + (jnp.arange(padn, dtype=jnp.int32) % (_NACC - _N))
        edges = jnp.concatenate([edge_index[0],
                                 jnp.zeros((padn,), jnp.int32),
                                 edge_index[1], pad_dst])
        dst_off = E + padn
    best = None
    for k0 in range(2, -(-nch // _NS) + 4, 2):
        rem = max(0, nch - _NS * k0)
        k1 = -(-rem // _NS)
        k1 += k1 & 1
        score = max(k0, k1)
        if best is None or score < best[0]:
            best = (score, k0, k1)
    _, k0, k1 = best
    seg = _make_seg(k0, k1, nch, dst_off)

    R = 1000
    NB = _N // R
    f32 = jnp.float32

    def blk():
        return pl.BlockSpec((R, _H), lambda i: (i, 0))

    wblk = pl.BlockSpec((_H, _H), lambda i: (0, 0))
    vblk = pl.BlockSpec((1, _H), lambda i: (0, 0))
    sblk = pl.BlockSpec((8, _H), lambda i: (0, 0))

    dense = pl.pallas_call(
        _dense, grid=(NB,),
        in_specs=[blk(), blk(), blk(), wblk, vblk, wblk, vblk],
        out_specs=[blk(), sblk],
        out_shape=[jax.ShapeDtypeStruct((_N, _H), f32),
                   jax.ShapeDtypeStruct((8, _H), f32)])
    bn = pl.pallas_call(
        _bn, grid=(NB,),
        in_specs=[blk(), sblk, vblk, vblk],
        out_specs=blk(),
        out_shape=jax.ShapeDtypeStruct((_N, _H), f32))
    readout = pl.pallas_call(
        _bn_readout, grid=(NB,),
        in_specs=[blk(), sblk, vblk, vblk, wblk, vblk],
        out_specs=[sblk, sblk],
        out_shape=[jax.ShapeDtypeStruct((8, _H), f32),
                   jax.ShapeDtypeStruct((8, _H), f32)])

    b1r, br1r = b1.reshape(1, _H), br1.reshape(1, _H)
    g1r, be1r = g1.reshape(1, _H), be1.reshape(1, _H)
    b2r, br2r = b2.reshape(1, _H), br2.reshape(1, _H)
    g2r, be2r = g2.reshape(1, _H), be2.reshape(1, _H)
    watt = jnp.broadcast_to(w_att, (_H, _H))
    batt = jnp.broadcast_to(b_att.reshape(1, 1), (1, _H))

    p1a, p1b = seg(x, edges)
    u1, st1 = dense(p1a, p1b, x, W1, b1r, Wr1, br1r)
    h1 = bn(u1, st1, g1r, be1r)
    p2a, p2b = seg(h1, edges)
    u2, st2 = dense(p2a, p2b, h1, W2, b2r, Wr2, br2r)
    s_out, m_out = readout(u2, st2, g2r, be2r, watt, batt)
    return jnp.concatenate([s_out[0:1], m_out[0:1]], axis=1)


# speedup vs baseline: 1.1651x; 1.1651x over previous
import jax
import jax.numpy as jnp
from jax import lax
from jax.experimental import pallas as pl
from jax.experimental.pallas import tpu as pltpu
from jax.experimental.pallas import tpu_sc as plsc

_N = 10000
_H = 128
_NC = 2
_NS = 16
_NW = _NC * _NS
_CHUNK = 128
_NACC = 10112
_ZR = _NACC // _NS


def _sc_segment_sum(k0, k1, nch, dst_off, h_hbm, edges_hbm, out0_hbm, out1_hbm,
                    srcb0, dstb0, srcb1, dstb1, rows0, rows1,
                    acc, semi0, semi1, semg0, semg1):
    c = lax.axis_index("c")
    s = lax.axis_index("s")
    is0 = c == 0
    my_k = jnp.where(is0, k0, k1)
    base = jnp.where(is0, s * k0, _NS * k0 + s * k1)
    my_k = jnp.minimum(my_k, jnp.maximum(nch - base, 0))

    def zbody(r, carry):
        for q in range(8):
            rows0[r, pl.ds(q * 16, 16)] = jnp.zeros((16,), jnp.float32)
        return carry

    lax.fori_loop(0, _CHUNK, zbody, 0)
    for t in range(4):
        pltpu.sync_copy(rows0, acc.at[pl.ds(s * _ZR + t * _CHUNK, _CHUNK)])
    pltpu.sync_copy(rows0.at[pl.ds(0, _ZR - 4 * _CHUNK)],
                    acc.at[pl.ds(s * _ZR + 4 * _CHUNK, _ZR - 4 * _CHUNK)])
    plsc.subcore_barrier()

    def sslice(g):
        return pl.ds(g * _CHUNK, _CHUNK)

    def dslice(g):
        return pl.ds(dst_off + g * _CHUNK, _CHUNK)

    @pl.when(my_k > 0)
    def _():
        pltpu.async_copy(edges_hbm.at[sslice(base)], srcb0, semi0)
        pltpu.async_copy(edges_hbm.at[dslice(base)], dstb0, semi0)
        pltpu.async_copy(edges_hbm.at[sslice(base + 1)], srcb1, semi1)
        pltpu.async_copy(edges_hbm.at[dslice(base + 1)], dstb1, semi1)

    def body(i, carry):
        a = 2 * i
        pltpu.make_async_copy(edges_hbm.at[sslice(base + a)], srcb0, semi0).wait()
        pltpu.make_async_copy(edges_hbm.at[dslice(base + a)], dstb0, semi0).wait()
        g0 = pltpu.async_copy(h_hbm.at[srcb0], rows0, semg0)
        pltpu.make_async_copy(edges_hbm.at[sslice(base + a + 1)], srcb1, semi1).wait()
        pltpu.make_async_copy(edges_hbm.at[dslice(base + a + 1)], dstb1, semi1).wait()
        g1 = pltpu.async_copy(h_hbm.at[srcb1], rows1, semg1)
        g0.wait()
        pltpu.sync_copy(rows0, acc.at[dstb0], add=True)

        @pl.when(a + 2 < my_k)
        def _():
            pltpu.async_copy(edges_hbm.at[sslice(base + a + 2)], srcb0, semi0)
            pltpu.async_copy(edges_hbm.at[dslice(base + a + 2)], dstb0, semi0)

        g1.wait()
        pltpu.sync_copy(rows1, acc.at[dstb1], add=True)

        @pl.when(a + 3 < my_k)
        def _():
            pltpu.async_copy(edges_hbm.at[sslice(base + a + 3)], srcb1, semi1)
            pltpu.async_copy(edges_hbm.at[dslice(base + a + 3)], dstb1, semi1)

        return carry

    lax.fori_loop(0, my_k // 2, body, 0)
    plsc.subcore_barrier()

    rpw = (_N // _NS) & ~7
    tail = _N - _NS * rpw

    @pl.when(is0)
    def _():
        pltpu.sync_copy(acc.at[pl.ds(s * rpw, rpw)],
                        out0_hbm.at[pl.ds(s * rpw, rpw)])

        @pl.when(s == 0)
        def _():
            pltpu.sync_copy(acc.at[pl.ds(_NS * rpw, tail)],
                            out0_hbm.at[pl.ds(_NS * rpw, tail)])

    @pl.when(jnp.logical_not(is0))
    def _():
        pltpu.sync_copy(acc.at[pl.ds(s * rpw, rpw)],
                        out1_hbm.at[pl.ds(s * rpw, rpw)])

        @pl.when(s == 0)
        def _():
            pltpu.sync_copy(acc.at[pl.ds(_NS * rpw, tail)],
                            out1_hbm.at[pl.ds(_NS * rpw, tail)])


def _make_seg(k0, k1, nch, dst_off):
    import functools
    mesh = plsc.VectorSubcoreMesh(core_axis_name="c", subcore_axis_name="s")
    return pl.kernel(
        functools.partial(_sc_segment_sum, k0, k1, nch, dst_off),
        mesh=mesh,
        out_type=[jax.ShapeDtypeStruct((_N, _H), jnp.float32),
                  jax.ShapeDtypeStruct((_N, _H), jnp.float32)],
        scratch_types=[
            pltpu.VMEM((_CHUNK,), jnp.int32),
            pltpu.VMEM((_CHUNK,), jnp.int32),
            pltpu.VMEM((_CHUNK,), jnp.int32),
            pltpu.VMEM((_CHUNK,), jnp.int32),
            pltpu.VMEM((_CHUNK, _H), jnp.float32),
            pltpu.VMEM((_CHUNK, _H), jnp.float32),
            pltpu.VMEM_SHARED((_NACC, _H), jnp.float32),
            pltpu.SemaphoreType.DMA,
            pltpu.SemaphoreType.DMA,
            pltpu.SemaphoreType.DMA,
            pltpu.SemaphoreType.DMA,
        ],
    )


def _dense(p0, p1, h, W, b, Wr, br, u_out, stats):
    i = pl.program_id(0)
    agg = p0[...] + p1[...]
    u = jnp.maximum(jnp.dot(agg, W[...], preferred_element_type=jnp.float32)
                    + b[...], 0.0)
    r = jnp.maximum(jnp.dot(h[...], Wr[...], preferred_element_type=jnp.float32)
                    + br[...], 0.0)
    u = u + r
    u_out[...] = u

    @pl.when(i == 0)
    def _():
        stats[...] = jnp.zeros_like(stats)

    stats[0:1, :] += jnp.sum(u, axis=0, keepdims=True)
    stats[1:2, :] += jnp.sum(u * u, axis=0, keepdims=True)


def _bn(u, stats, g, be, h_out):
    mu = stats[0:1, :] * (1.0 / _N)
    var = stats[1:2, :] * (1.0 / _N) - mu * mu
    sc = g[...] * lax.rsqrt(var + 1e-5)
    h_out[...] = (u[...] - mu) * sc + be[...]


def _bn_readout(u, stats, g, be, watt, batt, sum_out, max_out):
    i = pl.program_id(0)
    mu = stats[0:1, :] * (1.0 / _N)
    var = stats[1:2, :] * (1.0 / _N) - mu * mu
    sc = g[...] * lax.rsqrt(var + 1e-5)
    hh = (u[...] - mu) * sc + be[...]
    logits = jnp.dot(hh, watt[...], preferred_element_type=jnp.float32) + batt[...]
    w = jax.nn.sigmoid(logits[:, 0:1])
    ps = jnp.sum(w * hh, axis=0, keepdims=True)
    pm = jnp.max(hh, axis=0, keepdims=True)

    @pl.when(i == 0)
    def _():
        sum_out[...] = jnp.zeros_like(sum_out)
        max_out[...] = jnp.full_like(max_out, -jnp.inf)

    sum_out[0:1, :] += ps
    max_out[0:1, :] = jnp.maximum(max_out[0:1, :], pm)


def kernel(x, edge_index, W1, b1, Wr1, br1, g1, be1,
           W2, b2, Wr2, br2, g2, be2, w_att, b_att):
    E = edge_index.shape[1]
    nch = E // _CHUNK
    if nch * _CHUNK == E and nch % 2 == 0:
        edges = edge_index.reshape(2 * E)
        dst_off = E
    else:
        nch = -(-E // _CHUNK)
        nch += nch & 1
        padn = nch * _CHUNK - E
        pad_dst = _N + (jnp.arange(padn, dtype=jnp.int32) % (_NACC - _N))
        edges = jnp.concatenate([edge_index[0],
                                 jnp.zeros((padn,), jnp.int32),
                                 edge_index[1], pad_dst])
        dst_off = E + padn
    best = None
    for k0 in range(2, -(-nch // _NS) + 4, 2):
        rem = max(0, nch - _NS * k0)
        k1 = -(-rem // _NS)
        k1 += k1 & 1
        score = max(k0, k1)
        if best is None or score < best[0]:
            best = (score, k0, k1)
    _, k0, k1 = best
    seg = _make_seg(k0, k1, nch, dst_off)

    R = 1000
    NB = _N // R
    f32 = jnp.float32

    def blk():
        return pl.BlockSpec((R, _H), lambda i: (i, 0))

    wblk = pl.BlockSpec((_H, _H), lambda i: (0, 0))
    vblk = pl.BlockSpec((1, _H), lambda i: (0, 0))
    sblk = pl.BlockSpec((8, _H), lambda i: (0, 0))

    dense = pl.pallas_call(
        _dense, grid=(NB,),
        in_specs=[blk(), blk(), blk(), wblk, vblk, wblk, vblk],
        out_specs=[blk(), sblk],
        out_shape=[jax.ShapeDtypeStruct((_N, _H), f32),
                   jax.ShapeDtypeStruct((8, _H), f32)])
    bn = pl.pallas_call(
        _bn, grid=(NB,),
        in_specs=[blk(), sblk, vblk, vblk],
        out_specs=blk(),
        out_shape=jax.ShapeDtypeStruct((_N, _H), f32))
    readout = pl.pallas_call(
        _bn_readout, grid=(NB,),
        in_specs=[blk(), sblk, vblk, vblk, wblk, vblk],
        out_specs=[sblk, sblk],
        out_shape=[jax.ShapeDtypeStruct((8, _H), f32),
                   jax.ShapeDtypeStruct((8, _H), f32)])

    b1r, br1r = b1.reshape(1, _H), br1.reshape(1, _H)
    g1r, be1r = g1.reshape(1, _H), be1.reshape(1, _H)
    b2r, br2r = b2.reshape(1, _H), br2.reshape(1, _H)
    g2r, be2r = g2.reshape(1, _H), be2.reshape(1, _H)
    watt = jnp.broadcast_to(w_att, (_H, _H))
    batt = jnp.broadcast_to(b_att.reshape(1, 1), (1, _H))

    p1a, p1b = seg(x, edges)
    u1, st1 = dense(p1a, p1b, x, W1, b1r, Wr1, br1r)
    h1 = bn(u1, st1, g1r, be1r)
    p2a, p2b = seg(h1, edges)
    u2, st2 = dense(p2a, p2b, h1, W2, b2r, Wr2, br2r)
    s_out, m_out = readout(u2, st2, g2r, be2r, watt, batt)
    return jnp.concatenate([s_out[0:1], m_out[0:1]], axis=1)


# speedup vs baseline: 1.2009x; 1.0307x over previous
import jax
import jax.numpy as jnp
from jax import lax
from jax.experimental import pallas as pl
from jax.experimental.pallas import tpu as pltpu
from jax.experimental.pallas import tpu_sc as plsc

_N = 10000
_H = 128
_NC = 2
_NS = 16
_NW = _NC * _NS
_CHUNK = 128
_NACC = 10112
_ZR = _NACC // _NS


def _sc_segment_sum(k0, k1, nch, dst_off, h_hbm, edges_hbm, out0_hbm, out1_hbm,
                    srcb0, dstb0, srcb1, dstb1, rows0, rows1,
                    acc, semi0, semi1, semg0, semg1):
    c = lax.axis_index("c")
    s = lax.axis_index("s")
    is0 = c == 0
    my_k = jnp.where(is0, k0, k1)
    base = jnp.where(is0, s * k0, _NS * k0 + s * k1)
    my_k = jnp.minimum(my_k, jnp.maximum(nch - base, 0))

    def zbody(r, carry):
        for q in range(8):
            rows0[r, pl.ds(q * 16, 16)] = jnp.zeros((16,), jnp.float32)
        return carry

    lax.fori_loop(0, _CHUNK, zbody, 0)
    for t in range(4):
        pltpu.sync_copy(rows0, acc.at[pl.ds(s * _ZR + t * _CHUNK, _CHUNK)])
    pltpu.sync_copy(rows0.at[pl.ds(0, _ZR - 4 * _CHUNK)],
                    acc.at[pl.ds(s * _ZR + 4 * _CHUNK, _ZR - 4 * _CHUNK)])
    plsc.subcore_barrier()

    def sslice(g):
        return pl.ds(g * _CHUNK, _CHUNK)

    def dslice(g):
        return pl.ds(dst_off + g * _CHUNK, _CHUNK)

    @pl.when(my_k > 0)
    def _():
        pltpu.async_copy(edges_hbm.at[sslice(base)], srcb0, semi0)
        pltpu.async_copy(edges_hbm.at[dslice(base)], dstb0, semi0)
        pltpu.async_copy(edges_hbm.at[sslice(base + 1)], srcb1, semi1)
        pltpu.async_copy(edges_hbm.at[dslice(base + 1)], dstb1, semi1)

    def body(i, carry):
        a = 2 * i
        pltpu.make_async_copy(edges_hbm.at[sslice(base + a)], srcb0, semi0).wait()
        pltpu.make_async_copy(edges_hbm.at[dslice(base + a)], dstb0, semi0).wait()
        g0 = pltpu.async_copy(h_hbm.at[srcb0], rows0, semg0)
        pltpu.make_async_copy(edges_hbm.at[sslice(base + a + 1)], srcb1, semi1).wait()
        pltpu.make_async_copy(edges_hbm.at[dslice(base + a + 1)], dstb1, semi1).wait()
        g1 = pltpu.async_copy(h_hbm.at[srcb1], rows1, semg1)
        g0.wait()
        pltpu.sync_copy(rows0, acc.at[dstb0], add=True)

        @pl.when(a + 2 < my_k)
        def _():
            pltpu.async_copy(edges_hbm.at[sslice(base + a + 2)], srcb0, semi0)
            pltpu.async_copy(edges_hbm.at[dslice(base + a + 2)], dstb0, semi0)

        g1.wait()
        pltpu.sync_copy(rows1, acc.at[dstb1], add=True)

        @pl.when(a + 3 < my_k)
        def _():
            pltpu.async_copy(edges_hbm.at[sslice(base + a + 3)], srcb1, semi1)
            pltpu.async_copy(edges_hbm.at[dslice(base + a + 3)], dstb1, semi1)

        return carry

    lax.fori_loop(0, my_k // 2, body, 0)
    plsc.subcore_barrier()

    rpw = (_N // _NS) & ~7
    tail = _N - _NS * rpw

    @pl.when(is0)
    def _():
        pltpu.sync_copy(acc.at[pl.ds(s * rpw, rpw)],
                        out0_hbm.at[pl.ds(s * rpw, rpw)])

        @pl.when(s == 0)
        def _():
            pltpu.sync_copy(acc.at[pl.ds(_NS * rpw, tail)],
                            out0_hbm.at[pl.ds(_NS * rpw, tail)])

    @pl.when(jnp.logical_not(is0))
    def _():
        pltpu.sync_copy(acc.at[pl.ds(s * rpw, rpw)],
                        out1_hbm.at[pl.ds(s * rpw, rpw)])

        @pl.when(s == 0)
        def _():
            pltpu.sync_copy(acc.at[pl.ds(_NS * rpw, tail)],
                            out1_hbm.at[pl.ds(_NS * rpw, tail)])


def _make_seg(k0, k1, nch, dst_off):
    import functools
    mesh = plsc.VectorSubcoreMesh(core_axis_name="c", subcore_axis_name="s")
    return pl.kernel(
        functools.partial(_sc_segment_sum, k0, k1, nch, dst_off),
        mesh=mesh,
        out_type=[jax.ShapeDtypeStruct((_N, _H), jnp.float32),
                  jax.ShapeDtypeStruct((_N, _H), jnp.float32)],
        scratch_types=[
            pltpu.VMEM((_CHUNK,), jnp.int32),
            pltpu.VMEM((_CHUNK,), jnp.int32),
            pltpu.VMEM((_CHUNK,), jnp.int32),
            pltpu.VMEM((_CHUNK,), jnp.int32),
            pltpu.VMEM((_CHUNK, _H), jnp.float32),
            pltpu.VMEM((_CHUNK, _H), jnp.float32),
            pltpu.VMEM_SHARED((_NACC, _H), jnp.float32),
            pltpu.SemaphoreType.DMA,
            pltpu.SemaphoreType.DMA,
            pltpu.SemaphoreType.DMA,
            pltpu.SemaphoreType.DMA,
        ],
    )


def _dense(p0, p1, h, W, b, Wr, br, u_out, stats):
    i = pl.program_id(0)
    agg = p0[...] + p1[...]
    u = jnp.maximum(jnp.dot(agg, W[...], preferred_element_type=jnp.float32)
                    + b[...], 0.0)
    r = jnp.maximum(jnp.dot(h[...], Wr[...], preferred_element_type=jnp.float32)
                    + br[...], 0.0)
    u = u + r
    u_out[...] = u

    @pl.when(i == 0)
    def _():
        stats[...] = jnp.zeros_like(stats)

    stats[0:1, :] += jnp.sum(u, axis=0, keepdims=True)
    stats[1:2, :] += jnp.sum(u * u, axis=0, keepdims=True)


def _bn(u, stats, g, be, h_out):
    mu = stats[0:1, :] * (1.0 / _N)
    var = stats[1:2, :] * (1.0 / _N) - mu * mu
    sc = g[...] * lax.rsqrt(var + 1e-5)
    h_out[...] = (u[...] - mu) * sc + be[...]


def _bn_readout(u, stats, g, be, watt, batt, sum_out, max_out):
    i = pl.program_id(0)
    mu = stats[0:1, :] * (1.0 / _N)
    var = stats[1:2, :] * (1.0 / _N) - mu * mu
    sc = g[...] * lax.rsqrt(var + 1e-5)
    hh = (u[...] - mu) * sc + be[...]
    logits = jnp.dot(hh, watt[...], preferred_element_type=jnp.float32) + batt[...]
    w = jax.nn.sigmoid(logits[:, 0:1])
    ps = jnp.sum(w * hh, axis=0, keepdims=True)
    pm = jnp.max(hh, axis=0, keepdims=True)

    @pl.when(i == 0)
    def _():
        sum_out[...] = jnp.zeros_like(sum_out)
        max_out[...] = jnp.full_like(max_out, -jnp.inf)

    sum_out[0:1, :] += ps
    max_out[0:1, :] = jnp.maximum(max_out[0:1, :], pm)


def kernel(x, edge_index, W1, b1, Wr1, br1, g1, be1,
           W2, b2, Wr2, br2, g2, be2, w_att, b_att):
    E = edge_index.shape[1]
    nch = E // _CHUNK
    if nch * _CHUNK == E and nch % 2 == 0:
        edges = edge_index.reshape(2 * E)
        dst_off = E
    else:
        nch = -(-E // _CHUNK)
        nch += nch & 1
        padn = nch * _CHUNK - E
        pad_dst = _N + (jnp.arange(padn, dtype=jnp.int32) % (_NACC - _N))
        edges = jnp.concatenate([edge_index[0],
                                 jnp.zeros((padn,), jnp.int32),
                                 edge_index[1], pad_dst])
        dst_off = E + padn
    best = None
    for k0 in range(2, -(-nch // _NS) + 4, 2):
        rem = max(0, nch - _NS * k0)
        k1 = -(-rem // _NS)
        k1 += k1 & 1
        score = max(k0, k1)
        if best is None or score < best[0]:
            best = (score, k0, k1)
    _, k0, k1 = best
    seg = _make_seg(k0, k1, nch, dst_off)

    R = 2000
    NB = _N // R
    f32 = jnp.float32

    def blk():
        return pl.BlockSpec((R, _H), lambda i: (i, 0))

    wblk = pl.BlockSpec((_H, _H), lambda i: (0, 0))
    vblk = pl.BlockSpec((1, _H), lambda i: (0, 0))
    sblk = pl.BlockSpec((8, _H), lambda i: (0, 0))

    dense = pl.pallas_call(
        _dense, grid=(NB,),
        in_specs=[blk(), blk(), blk(), wblk, vblk, wblk, vblk],
        out_specs=[blk(), sblk],
        out_shape=[jax.ShapeDtypeStruct((_N, _H), f32),
                   jax.ShapeDtypeStruct((8, _H), f32)])
    bn = pl.pallas_call(
        _bn, grid=(NB,),
        in_specs=[blk(), sblk, vblk, vblk],
        out_specs=blk(),
        out_shape=jax.ShapeDtypeStruct((_N, _H), f32))
    readout = pl.pallas_call(
        _bn_readout, grid=(NB,),
        in_specs=[blk(), sblk, vblk, vblk, wblk, vblk],
        out_specs=[sblk, sblk],
        out_shape=[jax.ShapeDtypeStruct((8, _H), f32),
                   jax.ShapeDtypeStruct((8, _H), f32)])

    b1r, br1r = b1.reshape(1, _H), br1.reshape(1, _H)
    g1r, be1r = g1.reshape(1, _H), be1.reshape(1, _H)
    b2r, br2r = b2.reshape(1, _H), br2.reshape(1, _H)
    g2r, be2r = g2.reshape(1, _H), be2.reshape(1, _H)
    watt = jnp.broadcast_to(w_att, (_H, _H))
    batt = jnp.broadcast_to(b_att.reshape(1, 1), (1, _H))

    p1a, p1b = seg(x, edges)
    u1, st1 = dense(p1a, p1b, x, W1, b1r, Wr1, br1r)
    h1 = bn(u1, st1, g1r, be1r)
    p2a, p2b = seg(h1, edges)
    u2, st2 = dense(p2a, p2b, h1, W2, b2r, Wr2, br2r)
    s_out, m_out = readout(u2, st2, g2r, be2r, watt, batt)
    return jnp.concatenate([s_out[0:1], m_out[0:1]], axis=1)


# speedup vs baseline: 1.2195x; 1.0154x over previous
import jax
import jax.numpy as jnp
from jax import lax
from jax.experimental import pallas as pl
from jax.experimental.pallas import tpu as pltpu
from jax.experimental.pallas import tpu_sc as plsc

_N = 10000
_H = 128
_NC = 2
_NS = 16
_NW = _NC * _NS
_CHUNK = 128
_NACC = 10112
_ZR = _NACC // _NS


def _sc_segment_sum(k0, k1, nch, dst_off, h_hbm, edges_hbm, out0_hbm, out1_hbm,
                    srcb0, dstb0, srcb1, dstb1, rows0, rows1,
                    acc, semi0, semi1, semg0, semg1):
    c = lax.axis_index("c")
    s = lax.axis_index("s")
    is0 = c == 0
    my_k = jnp.where(is0, k0, k1)
    base = jnp.where(is0, s * k0, _NS * k0 + s * k1)
    my_k = jnp.minimum(my_k, jnp.maximum(nch - base, 0))

    def zbody(r, carry):
        for q in range(8):
            rows0[r, pl.ds(q * 16, 16)] = jnp.zeros((16,), jnp.float32)
        return carry

    lax.fori_loop(0, _CHUNK, zbody, 0)
    for t in range(4):
        pltpu.sync_copy(rows0, acc.at[pl.ds(s * _ZR + t * _CHUNK, _CHUNK)])
    pltpu.sync_copy(rows0.at[pl.ds(0, _ZR - 4 * _CHUNK)],
                    acc.at[pl.ds(s * _ZR + 4 * _CHUNK, _ZR - 4 * _CHUNK)])
    plsc.subcore_barrier()

    def sslice(g):
        return pl.ds(g * _CHUNK, _CHUNK)

    def dslice(g):
        return pl.ds(dst_off + g * _CHUNK, _CHUNK)

    @pl.when(my_k > 0)
    def _():
        pltpu.async_copy(edges_hbm.at[sslice(base)], srcb0, semi0)
        pltpu.async_copy(edges_hbm.at[dslice(base)], dstb0, semi0)
        pltpu.async_copy(edges_hbm.at[sslice(base + 1)], srcb1, semi1)
        pltpu.async_copy(edges_hbm.at[dslice(base + 1)], dstb1, semi1)

    def body(i, carry):
        a = 2 * i
        pltpu.make_async_copy(edges_hbm.at[sslice(base + a)], srcb0, semi0).wait()
        pltpu.make_async_copy(edges_hbm.at[dslice(base + a)], dstb0, semi0).wait()
        g0 = pltpu.async_copy(h_hbm.at[srcb0], rows0, semg0)
        pltpu.make_async_copy(edges_hbm.at[sslice(base + a + 1)], srcb1, semi1).wait()
        pltpu.make_async_copy(edges_hbm.at[dslice(base + a + 1)], dstb1, semi1).wait()
        g1 = pltpu.async_copy(h_hbm.at[srcb1], rows1, semg1)
        g0.wait()
        pltpu.sync_copy(rows0, acc.at[dstb0], add=True)

        @pl.when(a + 2 < my_k)
        def _():
            pltpu.async_copy(edges_hbm.at[sslice(base + a + 2)], srcb0, semi0)
            pltpu.async_copy(edges_hbm.at[dslice(base + a + 2)], dstb0, semi0)

        g1.wait()
        pltpu.sync_copy(rows1, acc.at[dstb1], add=True)

        @pl.when(a + 3 < my_k)
        def _():
            pltpu.async_copy(edges_hbm.at[sslice(base + a + 3)], srcb1, semi1)
            pltpu.async_copy(edges_hbm.at[dslice(base + a + 3)], dstb1, semi1)

        return carry

    lax.fori_loop(0, my_k // 2, body, 0)
    plsc.subcore_barrier()

    rpw = (_N // _NS) & ~7
    tail = _N - _NS * rpw

    @pl.when(is0)
    def _():
        pltpu.sync_copy(acc.at[pl.ds(s * rpw, rpw)],
                        out0_hbm.at[pl.ds(s * rpw, rpw)])

        @pl.when(s == 0)
        def _():
            pltpu.sync_copy(acc.at[pl.ds(_NS * rpw, tail)],
                            out0_hbm.at[pl.ds(_NS * rpw, tail)])

    @pl.when(jnp.logical_not(is0))
    def _():
        pltpu.sync_copy(acc.at[pl.ds(s * rpw, rpw)],
                        out1_hbm.at[pl.ds(s * rpw, rpw)])

        @pl.when(s == 0)
        def _():
            pltpu.sync_copy(acc.at[pl.ds(_NS * rpw, tail)],
                            out1_hbm.at[pl.ds(_NS * rpw, tail)])


def _make_seg(k0, k1, nch, dst_off):
    import functools
    mesh = plsc.VectorSubcoreMesh(core_axis_name="c", subcore_axis_name="s")
    return pl.kernel(
        functools.partial(_sc_segment_sum, k0, k1, nch, dst_off),
        mesh=mesh,
        out_type=[jax.ShapeDtypeStruct((_N, _H), jnp.float32),
                  jax.ShapeDtypeStruct((_N, _H), jnp.float32)],
        scratch_types=[
            pltpu.VMEM((_CHUNK,), jnp.int32),
            pltpu.VMEM((_CHUNK,), jnp.int32),
            pltpu.VMEM((_CHUNK,), jnp.int32),
            pltpu.VMEM((_CHUNK,), jnp.int32),
            pltpu.VMEM((_CHUNK, _H), jnp.float32),
            pltpu.VMEM((_CHUNK, _H), jnp.float32),
            pltpu.VMEM_SHARED((_NACC, _H), jnp.float32),
            pltpu.SemaphoreType.DMA,
            pltpu.SemaphoreType.DMA,
            pltpu.SemaphoreType.DMA,
            pltpu.SemaphoreType.DMA,
        ],
    )


def _dense(p0, p1, h, W, b, Wr, br, u_out, stats):
    i = pl.program_id(0)
    agg = p0[...] + p1[...]
    u = jnp.maximum(jnp.dot(agg, W[...], preferred_element_type=jnp.float32)
                    + b[...], 0.0)
    r = jnp.maximum(jnp.dot(h[...], Wr[...], preferred_element_type=jnp.float32)
                    + br[...], 0.0)
    u = u + r
    u_out[...] = u

    @pl.when(i == 0)
    def _():
        stats[...] = jnp.zeros_like(stats)

    stats[0:1, :] += jnp.sum(u, axis=0, keepdims=True)
    stats[1:2, :] += jnp.sum(u * u, axis=0, keepdims=True)


def _bn(u, stats, g, be, h_out):
    mu = stats[0:1, :] * (1.0 / _N)
    var = stats[1:2, :] * (1.0 / _N) - mu * mu
    sc = g[...] * lax.rsqrt(var + 1e-5)
    h_out[...] = (u[...] - mu) * sc + be[...]


def _bn_readout(u, stats, g, be, watt, batt, sum_out, max_out):
    i = pl.program_id(0)
    mu = stats[0:1, :] * (1.0 / _N)
    var = stats[1:2, :] * (1.0 / _N) - mu * mu
    sc = g[...] * lax.rsqrt(var + 1e-5)
    hh = (u[...] - mu) * sc + be[...]
    logits = jnp.dot(hh, watt[...], preferred_element_type=jnp.float32) + batt[...]
    w = jax.nn.sigmoid(logits[:, 0:1])
    ps = jnp.sum(w * hh, axis=0, keepdims=True)
    pm = jnp.max(hh, axis=0, keepdims=True)

    @pl.when(i == 0)
    def _():
        sum_out[...] = jnp.zeros_like(sum_out)
        max_out[...] = jnp.full_like(max_out, -jnp.inf)

    sum_out[0:1, :] += ps
    max_out[0:1, :] = jnp.maximum(max_out[0:1, :], pm)


def kernel(x, edge_index, W1, b1, Wr1, br1, g1, be1,
           W2, b2, Wr2, br2, g2, be2, w_att, b_att):
    E = edge_index.shape[1]
    nch = E // _CHUNK
    if nch * _CHUNK == E and nch % 2 == 0:
        edges = edge_index.reshape(2 * E)
        dst_off = E
    else:
        nch = -(-E // _CHUNK)
        nch += nch & 1
        padn = nch * _CHUNK - E
        pad_dst = _N + (jnp.arange(padn, dtype=jnp.int32) % (_NACC - _N))
        edges = jnp.concatenate([edge_index[0],
                                 jnp.zeros((padn,), jnp.int32),
                                 edge_index[1], pad_dst])
        dst_off = E + padn
    best = None
    for k0 in range(2, -(-nch // _NS) + 4, 2):
        rem = max(0, nch - _NS * k0)
        k1 = -(-rem // _NS)
        k1 += k1 & 1
        score = max(k0, k1)
        if best is None or score < best[0]:
            best = (score, k0, k1)
    _, k0, k1 = best
    seg = _make_seg(k0, k1, nch, dst_off)

    R = 5000
    NB = _N // R
    f32 = jnp.float32

    def blk():
        return pl.BlockSpec((R, _H), lambda i: (i, 0))

    wblk = pl.BlockSpec((_H, _H), lambda i: (0, 0))
    vblk = pl.BlockSpec((1, _H), lambda i: (0, 0))
    sblk = pl.BlockSpec((8, _H), lambda i: (0, 0))

    dense = pl.pallas_call(
        _dense, grid=(NB,),
        in_specs=[blk(), blk(), blk(), wblk, vblk, wblk, vblk],
        out_specs=[blk(), sblk],
        out_shape=[jax.ShapeDtypeStruct((_N, _H), f32),
                   jax.ShapeDtypeStruct((8, _H), f32)])
    bn = pl.pallas_call(
        _bn, grid=(NB,),
        in_specs=[blk(), sblk, vblk, vblk],
        out_specs=blk(),
        out_shape=jax.ShapeDtypeStruct((_N, _H), f32))
    readout = pl.pallas_call(
        _bn_readout, grid=(NB,),
        in_specs=[blk(), sblk, vblk, vblk, wblk, vblk],
        out_specs=[sblk, sblk],
        out_shape=[jax.ShapeDtypeStruct((8, _H), f32),
                   jax.ShapeDtypeStruct((8, _H), f32)])

    b1r, br1r = b1.reshape(1, _H), br1.reshape(1, _H)
    g1r, be1r = g1.reshape(1, _H), be1.reshape(1, _H)
    b2r, br2r = b2.reshape(1, _H), br2.reshape(1, _H)
    g2r, be2r = g2.reshape(1, _H), be2.reshape(1, _H)
    watt = jnp.broadcast_to(w_att, (_H, _H))
    batt = jnp.broadcast_to(b_att.reshape(1, 1), (1, _H))

    p1a, p1b = seg(x, edges)
    u1, st1 = dense(p1a, p1b, x, W1, b1r, Wr1, br1r)
    h1 = bn(u1, st1, g1r, be1r)
    p2a, p2b = seg(h1, edges)
    u2, st2 = dense(p2a, p2b, h1, W2, b2r, Wr2, br2r)
    s_out, m_out = readout(u2, st2, g2r, be2r, watt, batt)
    return jnp.concatenate([s_out[0:1], m_out[0:1]], axis=1)
